# baseline (speedup 1.0000x reference)
import jax
import jax.numpy as jnp
from jax.experimental import pallas as pl

_TARGET_ENTROPY = 8.0
_NB = 4096
_B = 1024
_D = 16
_K = 100000
_ROWS = 32
_CHUNK = 2048
_NFULL = _K // _CHUNK
_TAIL0 = _NFULL * _CHUNK
_BITERS = 16
_M = 14
_TAILW = _K - _TAIL0
_EPS_H = 4095e-7


def _fused_body(pts_ref, pidx_ref, bank_ref, sims_ref, acc_ref):
    i = pl.program_id(0)
    f32 = jnp.float32

    x = pts_ref[...]
    xn = x * jax.lax.rsqrt(jnp.sum(x * x, axis=1, keepdims=True))

    def _mm(w):
        wn = w * jax.lax.rsqrt(jnp.sum(w * w, axis=0, keepdims=True))
        return jax.lax.dot_general(
            xn, wn, (((1,), (0,)), ((), ())), preferred_element_type=f32
        )

    def _mm_chunk(c, carry):
        sims_ref[:, pl.ds(c * _CHUNK, _CHUNK)] = _mm(
            bank_ref[:, pl.ds(c * _CHUNK, _CHUNK)]
        )
        return carry

    jax.lax.fori_loop(0, _NFULL, _mm_chunk, 0)
    sims_ref[:, _TAIL0:] = _mm(bank_ref[:, _TAIL0:])

    p = pidx_ref[...]
    kt_p = p + 1.0

    ones_c = jnp.ones((_CHUNK, 8), f32)
    ones_t = jnp.ones((_TAILW, 8), f32)

    def _red(x, ones):
        return jax.lax.dot_general(
            x, ones, (((1,), (0,)), ((), ())), preferred_element_type=f32
        )

    def _counts(mid_k, mid_p):
        def body(c, carry):
            ck, cp = carry
            v = sims_ref[:, pl.ds(c * _CHUNK, _CHUNK)]
            ck = ck + _red(jnp.where(v > mid_k, 1.0, 0.0), ones_c)
            cp = cp + _red(jnp.where(v > mid_p, 1.0, 0.0), ones_c)
            return ck, cp
        z = jnp.zeros((_ROWS, 8), f32)
        ck, cp = jax.lax.fori_loop(0, _NFULL, body, (z, z))
        v = sims_ref[:, _TAIL0:]
        ck = ck + _red(jnp.where(v > mid_k, 1.0, 0.0), ones_t)
        cp = cp + _red(jnp.where(v > mid_p, 1.0, 0.0), ones_t)
        return ck[:, :1], cp[:, :1]

    def _bis(_, st):
        lo_k, hi_k, n_k, lo_p, hi_p = st
        mid_k = 0.5 * (lo_k + hi_k)
        mid_p = 0.5 * (lo_p + hi_p)
        ck, cp = _counts(mid_k, mid_p)
        ge_k = ck >= float(_NB)
        n_k = jnp.where(ge_k, ck, n_k)
        lo_k = jnp.where(ge_k, mid_k, lo_k)
        hi_k = jnp.where(ge_k, hi_k, mid_k)
        ge_p = cp >= kt_p
        lo_p = jnp.where(ge_p, mid_p, lo_p)
        hi_p = jnp.where(ge_p, hi_p, mid_p)
        return lo_k, hi_k, n_k, lo_p, hi_p

    ones = jnp.ones((_ROWS, 1), f32)
    st0 = (-1.001 * ones, 1.001 * ones, float(_K) * ones, -1.001 * ones, 1.001 * ones)
    lo_k, _, n_k, v_p, _ = jax.lax.fori_loop(0, _BITERS, _bis, st0)

    def _p1_body(c, s):
        v = sims_ref[:, pl.ds(c * _CHUNK, _CHUNK)]
        return s + _red(jnp.where(v > lo_k, v, 0.0), ones_c)

    p1 = jax.lax.fori_loop(0, _NFULL, _p1_body, jnp.zeros((_ROWS, 8), f32))
    v = sims_ref[:, _TAIL0:]
    p1 = (p1 + _red(jnp.where(v > lo_k, v, 0.0), ones_t))[:, :1]

    excess = n_k - float(_NB)
    vbar = (p1 - excess * lo_k) / float(_NB)

    def _mom(v, ones):
        d = jnp.where(v > lo_k, v - vbar, 0.0)
        cur = d
        out = []
        for m in range(1, _M + 1):
            out.append(_red(cur, ones))
            if m < _M:
                cur = cur * d
        return tuple(out)

    def _mom_body(c, carry):
        part = _mom(sims_ref[:, pl.ds(c * _CHUNK, _CHUNK)], ones_c)
        return tuple(a + b for a, b in zip(carry, part))

    z14 = tuple(jnp.zeros((_ROWS, 8), f32) for _ in range(_M))
    cs = jax.lax.fori_loop(0, _NFULL, _mom_body, z14)
    cs = tuple(
        (a + b)[:, :1]
        for a, b in zip(cs, _mom(sims_ref[:, _TAIL0:], ones_t))
    )
    dlo = lo_k - vbar
    corr = dlo
    cs_c = [None] * (_M + 1)
    cs_c[0] = float(_NB) * jnp.ones((_ROWS, 1), f32)
    for m in range(1, _M + 1):
        cs_c[m] = cs[m - 1] - excess * corr
        corr = corr * dlo

    def _sm(u):
        t0 = jnp.zeros((_ROWS, 1), f32)
        t1 = jnp.zeros((_ROWS, 1), f32)
        cm = jnp.ones((_ROWS, 1), f32)
        for m in range(_M + 1):
            t0 = t0 + cm * cs_c[m]
            if m < _M:
                t1 = t1 + cm * cs_c[m + 1]
            cm = cm * u / float(m + 1)
        e = jnp.exp(u * vbar)
        return e * t0, e * (vbar * t0 + t1)

    def _entropy(u):
        s, mv = _sm(u)
        ep = jnp.exp(u * v_p)
        sp = s - ep
        mp = mv - v_p * ep
        return jnp.log(sp) - u * mp / sp - _EPS_H

    centers = 5.0 * jnp.ones((_ROWS, 1), f32)
    scale = 2.5
    for _ in range(13):
        h = _entropy(1.0 / centers)
        ind = 2.0 * jnp.where(h < _TARGET_ENTROPY, 1.0, 0.0) - 1.0
        centers = centers + scale * ind
        scale = scale * 0.5
    u_f = 1.0 / centers
    h_f = _entropy(u_f)

    pos = jnp.zeros((_ROWS, 8), f32)
    for j in range(_NB // _CHUNK):
        v = sims_ref[:, j * _CHUNK:(j + 1) * _CHUNK]
        lane = jax.lax.broadcasted_iota(jnp.int32, (_ROWS, _CHUNK), 1).astype(f32)
        pos = pos + _red(jnp.where(lane == (p - float(j * _CHUNK)), v, 0.0), ones_c)
    pos = pos[:, :1]

    s_f, _ = _sm(u_f)
    denom = jnp.exp(-u_f) * s_f
    cond = jnp.exp((pos - 1.0) * u_f) / denom
    ll = jnp.log(cond + 1e-7)

    vec = jnp.concatenate(
        [jnp.sum(ll, axis=0, keepdims=True),
         jnp.sum(centers, axis=0, keepdims=True),
         jnp.sum(h_f, axis=0, keepdims=True)], axis=1)
    acc_ref[...] = jnp.where(i == 0, vec, acc_ref[...] + vec)


def kernel(points, point_indices, memory_bank):
    bank_t = memory_bank.T
    pidx_f = point_indices.astype(jnp.float32).reshape(_B, 1)
    sims, acc = pl.pallas_call(
        _fused_body,
        grid=(_B // _ROWS,),
        in_specs=[
            pl.BlockSpec((_ROWS, _D), lambda i: (i, 0)),
            pl.BlockSpec((_ROWS, 1), lambda i: (i, 0)),
            pl.BlockSpec((_D, _K), lambda i: (0, 0)),
        ],
        out_specs=[
            pl.BlockSpec((_ROWS, _K), lambda i: (i, 0)),
            pl.BlockSpec((1, 3), lambda i: (0, 0)),
        ],
        out_shape=[
            jax.ShapeDtypeStruct((_B, _K), jnp.float32),
            jax.ShapeDtypeStruct((1, 3), jnp.float32),
        ],
    )(points, pidx_f, bank_t)
    inv_b = 1.0 / float(_B)
    loss = -acc[0, 0] * inv_b
    return loss, sims, acc[0, 1] * inv_b, acc[0, 2] * inv_b


# speedup vs baseline: 1.8852x; 1.8852x over previous
import jax
import jax.numpy as jnp
from jax.experimental import pallas as pl

_TARGET_ENTROPY = 8.0
_NB = 4096
_B = 1024
_D = 16
_K = 100000
_ROWS = 32
_CHUNK = 2048
_NFULL = _K // _CHUNK
_TAIL0 = _NFULL * _CHUNK
_BITERS = 16
_M = 14
_TAILW = _K - _TAIL0
_EPS_H = 4095e-7


def _fused_body(pts_ref, pidx_ref, bank_ref, sims_ref, acc_ref):
    i = pl.program_id(0)
    f32 = jnp.float32

    x = pts_ref[...]
    xn = x * jax.lax.rsqrt(jnp.sum(x * x, axis=1, keepdims=True))

    def _mm(w):
        wn = w * jax.lax.rsqrt(jnp.sum(w * w, axis=0, keepdims=True))
        return jax.lax.dot_general(
            xn, wn, (((1,), (0,)), ((), ())), preferred_element_type=f32
        )

    def _mm_chunk(c, carry):
        sims_ref[:, pl.ds(c * _CHUNK, _CHUNK)] = _mm(
            bank_ref[:, pl.ds(c * _CHUNK, _CHUNK)]
        )
        return carry

    jax.lax.fori_loop(0, _NFULL, _mm_chunk, 0)
    sims_ref[:, _TAIL0:] = _mm(bank_ref[:, _TAIL0:])

    p = pidx_ref[...]
    kt_p = p + 1.0

    ones_c = jnp.ones((_CHUNK, 8), f32)
    ones_t = jnp.ones((_TAILW, 8), f32)

    def _red(x, ones):
        return jnp.sum(x, axis=1, keepdims=True)

    def _counts(mid_k, mid_p):
        def body(c, carry):
            ck, cp = carry
            v = sims_ref[:, pl.ds(c * _CHUNK, _CHUNK)]
            ck = ck + _red(jnp.where(v > mid_k, 1.0, 0.0), ones_c)
            cp = cp + _red(jnp.where(v > mid_p, 1.0, 0.0), ones_c)
            return ck, cp
        z = jnp.zeros((_ROWS, 1), f32)
        ck, cp = jax.lax.fori_loop(0, _NFULL, body, (z, z))
        v = sims_ref[:, _TAIL0:]
        ck = ck + _red(jnp.where(v > mid_k, 1.0, 0.0), ones_t)
        cp = cp + _red(jnp.where(v > mid_p, 1.0, 0.0), ones_t)
        return ck, cp

    def _bis(_, st):
        lo_k, hi_k, n_k, lo_p, hi_p = st
        mid_k = 0.5 * (lo_k + hi_k)
        mid_p = 0.5 * (lo_p + hi_p)
        ck, cp = _counts(mid_k, mid_p)
        ge_k = ck >= float(_NB)
        n_k = jnp.where(ge_k, ck, n_k)
        lo_k = jnp.where(ge_k, mid_k, lo_k)
        hi_k = jnp.where(ge_k, hi_k, mid_k)
        ge_p = cp >= kt_p
        lo_p = jnp.where(ge_p, mid_p, lo_p)
        hi_p = jnp.where(ge_p, hi_p, mid_p)
        return lo_k, hi_k, n_k, lo_p, hi_p

    ones = jnp.ones((_ROWS, 1), f32)
    st0 = (-1.001 * ones, 1.001 * ones, float(_K) * ones, -1.001 * ones, 1.001 * ones)
    lo_k, _, n_k, v_p, _ = jax.lax.fori_loop(0, _BITERS, _bis, st0)

    def _p1_body(c, s):
        v = sims_ref[:, pl.ds(c * _CHUNK, _CHUNK)]
        return s + _red(jnp.where(v > lo_k, v, 0.0), ones_c)

    p1 = jax.lax.fori_loop(0, _NFULL, _p1_body, jnp.zeros((_ROWS, 1), f32))
    v = sims_ref[:, _TAIL0:]
    p1 = p1 + _red(jnp.where(v > lo_k, v, 0.0), ones_t)

    excess = n_k - float(_NB)
    vbar = (p1 - excess * lo_k) / float(_NB)

    def _mom(v, ones):
        d = jnp.where(v > lo_k, v - vbar, 0.0)
        cur = d
        out = []
        for m in range(1, _M + 1):
            out.append(_red(cur, ones))
            if m < _M:
                cur = cur * d
        return tuple(out)

    def _mom_body(c, carry):
        part = _mom(sims_ref[:, pl.ds(c * _CHUNK, _CHUNK)], ones_c)
        return tuple(a + b for a, b in zip(carry, part))

    z14 = tuple(jnp.zeros((_ROWS, 1), f32) for _ in range(_M))
    cs = jax.lax.fori_loop(0, _NFULL, _mom_body, z14)
    cs = tuple(a + b for a, b in zip(cs, _mom(sims_ref[:, _TAIL0:], ones_t)))
    dlo = lo_k - vbar
    corr = dlo
    cs_c = [None] * (_M + 1)
    cs_c[0] = float(_NB) * jnp.ones((_ROWS, 1), f32)
    for m in range(1, _M + 1):
        cs_c[m] = cs[m - 1] - excess * corr
        corr = corr * dlo

    def _sm(u):
        t0 = jnp.zeros((_ROWS, 1), f32)
        t1 = jnp.zeros((_ROWS, 1), f32)
        cm = jnp.ones((_ROWS, 1), f32)
        for m in range(_M + 1):
            t0 = t0 + cm * cs_c[m]
            if m < _M:
                t1 = t1 + cm * cs_c[m + 1]
            cm = cm * u / float(m + 1)
        e = jnp.exp(u * vbar)
        return e * t0, e * (vbar * t0 + t1)

    def _entropy(u):
        s, mv = _sm(u)
        ep = jnp.exp(u * v_p)
        sp = s - ep
        mp = mv - v_p * ep
        return jnp.log(sp) - u * mp / sp - _EPS_H

    centers = 5.0 * jnp.ones((_ROWS, 1), f32)
    scale = 2.5
    for _ in range(13):
        h = _entropy(1.0 / centers)
        ind = 2.0 * jnp.where(h < _TARGET_ENTROPY, 1.0, 0.0) - 1.0
        centers = centers + scale * ind
        scale = scale * 0.5
    u_f = 1.0 / centers
    h_f = _entropy(u_f)

    pos = jnp.zeros((_ROWS, 1), f32)
    for j in range(_NB // _CHUNK):
        v = sims_ref[:, j * _CHUNK:(j + 1) * _CHUNK]
        lane = jax.lax.broadcasted_iota(jnp.int32, (_ROWS, _CHUNK), 1).astype(f32)
        pos = pos + _red(jnp.where(lane == (p - float(j * _CHUNK)), v, 0.0), ones_c)

    s_f, _ = _sm(u_f)
    denom = jnp.exp(-u_f) * s_f
    cond = jnp.exp((pos - 1.0) * u_f) / denom
    ll = jnp.log(cond + 1e-7)

    vec = jnp.concatenate(
        [jnp.sum(ll, axis=0, keepdims=True),
         jnp.sum(centers, axis=0, keepdims=True),
         jnp.sum(h_f, axis=0, keepdims=True)], axis=1)
    acc_ref[...] = jnp.where(i == 0, vec, acc_ref[...] + vec)


def kernel(points, point_indices, memory_bank):
    bank_t = memory_bank.T
    pidx_f = point_indices.astype(jnp.float32).reshape(_B, 1)
    sims, acc = pl.pallas_call(
        _fused_body,
        grid=(_B // _ROWS,),
        in_specs=[
            pl.BlockSpec((_ROWS, _D), lambda i: (i, 0)),
            pl.BlockSpec((_ROWS, 1), lambda i: (i, 0)),
            pl.BlockSpec((_D, _K), lambda i: (0, 0)),
        ],
        out_specs=[
            pl.BlockSpec((_ROWS, _K), lambda i: (i, 0)),
            pl.BlockSpec((1, 3), lambda i: (0, 0)),
        ],
        out_shape=[
            jax.ShapeDtypeStruct((_B, _K), jnp.float32),
            jax.ShapeDtypeStruct((1, 3), jnp.float32),
        ],
    )(points, pidx_f, bank_t)
    inv_b = 1.0 / float(_B)
    loss = -acc[0, 0] * inv_b
    return loss, sims, acc[0, 1] * inv_b, acc[0, 2] * inv_b


# speedup vs baseline: 2.7442x; 1.4556x over previous
import jax
import jax.numpy as jnp
from jax.experimental import pallas as pl

_TARGET_ENTROPY = 8.0
_NB = 4096
_B = 1024
_D = 16
_K = 100000
_ROWS = 32
_CHUNK = 4096
_NFULL = _K // _CHUNK
_TAIL0 = _NFULL * _CHUNK
_BITERS = 16
_M = 14
_TAILW = _K - _TAIL0
_EPS_H = 4095e-7


def _fused_body(pts_ref, pidx_ref, bank_ref, sims_ref, acc_ref):
    i = pl.program_id(0)
    f32 = jnp.float32

    x = pts_ref[...]
    xn = x * jax.lax.rsqrt(jnp.sum(x * x, axis=1, keepdims=True))

    def _mm(w):
        wn = w * jax.lax.rsqrt(jnp.sum(w * w, axis=0, keepdims=True))
        return jax.lax.dot_general(
            xn, wn, (((1,), (0,)), ((), ())), preferred_element_type=f32
        )

    def _mm_chunk(c, carry):
        sims_ref[:, pl.ds(c * _CHUNK, _CHUNK)] = _mm(
            bank_ref[:, pl.ds(c * _CHUNK, _CHUNK)]
        )
        return carry

    jax.lax.fori_loop(0, _NFULL, _mm_chunk, 0)
    sims_ref[:, _TAIL0:] = _mm(bank_ref[:, _TAIL0:])

    p = pidx_ref[...]
    kt_p = p + 1.0

    ones_c = jnp.ones((_CHUNK, 8), f32)
    ones_t = jnp.ones((_TAILW, 8), f32)

    def _red(x, ones):
        return jnp.sum(x, axis=1, keepdims=True)

    def _counts(mid_k, mid_p):
        def body(c, carry):
            ck, cp = carry
            v = sims_ref[:, pl.ds(c * _CHUNK, _CHUNK)]
            ck = ck + _red(jnp.where(v > mid_k, 1.0, 0.0), ones_c)
            cp = cp + _red(jnp.where(v > mid_p, 1.0, 0.0), ones_c)
            return ck, cp
        z = jnp.zeros((_ROWS, 1), f32)
        ck, cp = jax.lax.fori_loop(0, _NFULL, body, (z, z))
        v = sims_ref[:, _TAIL0:]
        ck = ck + _red(jnp.where(v > mid_k, 1.0, 0.0), ones_t)
        cp = cp + _red(jnp.where(v > mid_p, 1.0, 0.0), ones_t)
        return ck, cp

    def _bis(_, st):
        lo_k, hi_k, n_k, lo_p, hi_p = st
        mid_k = 0.5 * (lo_k + hi_k)
        mid_p = 0.5 * (lo_p + hi_p)
        ck, cp = _counts(mid_k, mid_p)
        ge_k = ck >= float(_NB)
        n_k = jnp.where(ge_k, ck, n_k)
        lo_k = jnp.where(ge_k, mid_k, lo_k)
        hi_k = jnp.where(ge_k, hi_k, mid_k)
        ge_p = cp >= kt_p
        lo_p = jnp.where(ge_p, mid_p, lo_p)
        hi_p = jnp.where(ge_p, hi_p, mid_p)
        return lo_k, hi_k, n_k, lo_p, hi_p

    ones = jnp.ones((_ROWS, 1), f32)
    st0 = (-1.001 * ones, 1.001 * ones, float(_K) * ones, -1.001 * ones, 1.001 * ones)
    lo_k, _, n_k, v_p, _ = jax.lax.fori_loop(0, _BITERS, _bis, st0)

    def _p1_body(c, s):
        v = sims_ref[:, pl.ds(c * _CHUNK, _CHUNK)]
        return s + _red(jnp.where(v > lo_k, v, 0.0), ones_c)

    p1 = jax.lax.fori_loop(0, _NFULL, _p1_body, jnp.zeros((_ROWS, 1), f32))
    v = sims_ref[:, _TAIL0:]
    p1 = p1 + _red(jnp.where(v > lo_k, v, 0.0), ones_t)

    excess = n_k - float(_NB)
    vbar = (p1 - excess * lo_k) / float(_NB)

    def _mom(v, ones):
        d = jnp.where(v > lo_k, v - vbar, 0.0)
        cur = d
        out = []
        for m in range(1, _M + 1):
            out.append(_red(cur, ones))
            if m < _M:
                cur = cur * d
        return tuple(out)

    def _mom_body(c, carry):
        part = _mom(sims_ref[:, pl.ds(c * _CHUNK, _CHUNK)], ones_c)
        return tuple(a + b for a, b in zip(carry, part))

    z14 = tuple(jnp.zeros((_ROWS, 1), f32) for _ in range(_M))
    cs = jax.lax.fori_loop(0, _NFULL, _mom_body, z14)
    cs = tuple(a + b for a, b in zip(cs, _mom(sims_ref[:, _TAIL0:], ones_t)))
    dlo = lo_k - vbar
    corr = dlo
    cs_c = [None] * (_M + 1)
    cs_c[0] = float(_NB) * jnp.ones((_ROWS, 1), f32)
    for m in range(1, _M + 1):
        cs_c[m] = cs[m - 1] - excess * corr
        corr = corr * dlo

    def _sm(u):
        t0 = jnp.zeros((_ROWS, 1), f32)
        t1 = jnp.zeros((_ROWS, 1), f32)
        cm = jnp.ones((_ROWS, 1), f32)
        for m in range(_M + 1):
            t0 = t0 + cm * cs_c[m]
            if m < _M:
                t1 = t1 + cm * cs_c[m + 1]
            cm = cm * u / float(m + 1)
        e = jnp.exp(u * vbar)
        return e * t0, e * (vbar * t0 + t1)

    def _entropy(u):
        s, mv = _sm(u)
        ep = jnp.exp(u * v_p)
        sp = s - ep
        mp = mv - v_p * ep
        return jnp.log(sp) - u * mp / sp - _EPS_H

    centers = 5.0 * jnp.ones((_ROWS, 1), f32)
    scale = 2.5
    for _ in range(13):
        h = _entropy(1.0 / centers)
        ind = 2.0 * jnp.where(h < _TARGET_ENTROPY, 1.0, 0.0) - 1.0
        centers = centers + scale * ind
        scale = scale * 0.5
    u_f = 1.0 / centers
    h_f = _entropy(u_f)

    pos = jnp.zeros((_ROWS, 1), f32)
    for j in range(_NB // _CHUNK):
        v = sims_ref[:, j * _CHUNK:(j + 1) * _CHUNK]
        lane = jax.lax.broadcasted_iota(jnp.int32, (_ROWS, _CHUNK), 1).astype(f32)
        pos = pos + _red(jnp.where(lane == (p - float(j * _CHUNK)), v, 0.0), ones_c)

    s_f, _ = _sm(u_f)
    denom = jnp.exp(-u_f) * s_f
    cond = jnp.exp((pos - 1.0) * u_f) / denom
    ll = jnp.log(cond + 1e-7)

    vec = jnp.concatenate(
        [jnp.sum(ll, axis=0, keepdims=True),
         jnp.sum(centers, axis=0, keepdims=True),
         jnp.sum(h_f, axis=0, keepdims=True)], axis=1)
    acc_ref[...] = jnp.where(i == 0, vec, acc_ref[...] + vec)


def kernel(points, point_indices, memory_bank):
    bank_t = memory_bank.T
    pidx_f = point_indices.astype(jnp.float32).reshape(_B, 1)
    sims, acc = pl.pallas_call(
        _fused_body,
        grid=(_B // _ROWS,),
        in_specs=[
            pl.BlockSpec((_ROWS, _D), lambda i: (i, 0)),
            pl.BlockSpec((_ROWS, 1), lambda i: (i, 0)),
            pl.BlockSpec((_D, _K), lambda i: (0, 0)),
        ],
        out_specs=[
            pl.BlockSpec((_ROWS, _K), lambda i: (i, 0)),
            pl.BlockSpec((1, 3), lambda i: (0, 0)),
        ],
        out_shape=[
            jax.ShapeDtypeStruct((_B, _K), jnp.float32),
            jax.ShapeDtypeStruct((1, 3), jnp.float32),
        ],
    )(points, pidx_f, bank_t)
    inv_b = 1.0 / float(_B)
    loss = -acc[0, 0] * inv_b
    return loss, sims, acc[0, 1] * inv_b, acc[0, 2] * inv_b


# speedup vs baseline: 3.4394x; 1.2533x over previous
import jax
import jax.numpy as jnp
from jax.experimental import pallas as pl

_TARGET_ENTROPY = 8.0
_NB = 4096
_B = 1024
_D = 16
_K = 100000
_ROWS = 32
_CHUNK = 8192
_NFULL = _K // _CHUNK
_TAIL0 = _NFULL * _CHUNK
_BITERS = 16
_M = 14
_TAILW = _K - _TAIL0
_EPS_H = 4095e-7


def _fused_body(pts_ref, pidx_ref, bank_ref, sims_ref, acc_ref):
    i = pl.program_id(0)
    f32 = jnp.float32

    x = pts_ref[...]
    xn = x * jax.lax.rsqrt(jnp.sum(x * x, axis=1, keepdims=True))

    def _mm(w):
        wn = w * jax.lax.rsqrt(jnp.sum(w * w, axis=0, keepdims=True))
        return jax.lax.dot_general(
            xn, wn, (((1,), (0,)), ((), ())), preferred_element_type=f32
        )

    def _mm_chunk(c, carry):
        sims_ref[:, pl.ds(c * _CHUNK, _CHUNK)] = _mm(
            bank_ref[:, pl.ds(c * _CHUNK, _CHUNK)]
        )
        return carry

    jax.lax.fori_loop(0, _NFULL, _mm_chunk, 0)
    sims_ref[:, _TAIL0:] = _mm(bank_ref[:, _TAIL0:])

    p = pidx_ref[...]
    kt_p = p + 1.0

    ones_c = jnp.ones((_CHUNK, 8), f32)
    ones_t = jnp.ones((_TAILW, 8), f32)

    def _red(x, ones):
        return jnp.sum(x, axis=1, keepdims=True)

    def _counts(mid_k, mid_p):
        def body(c, carry):
            ck, cp = carry
            v = sims_ref[:, pl.ds(c * _CHUNK, _CHUNK)]
            ck = ck + _red(jnp.where(v > mid_k, 1.0, 0.0), ones_c)
            cp = cp + _red(jnp.where(v > mid_p, 1.0, 0.0), ones_c)
            return ck, cp
        z = jnp.zeros((_ROWS, 1), f32)
        ck, cp = jax.lax.fori_loop(0, _NFULL, body, (z, z))
        v = sims_ref[:, _TAIL0:]
        ck = ck + _red(jnp.where(v > mid_k, 1.0, 0.0), ones_t)
        cp = cp + _red(jnp.where(v > mid_p, 1.0, 0.0), ones_t)
        return ck, cp

    def _bis(_, st):
        lo_k, hi_k, n_k, lo_p, hi_p = st
        mid_k = 0.5 * (lo_k + hi_k)
        mid_p = 0.5 * (lo_p + hi_p)
        ck, cp = _counts(mid_k, mid_p)
        ge_k = ck >= float(_NB)
        n_k = jnp.where(ge_k, ck, n_k)
        lo_k = jnp.where(ge_k, mid_k, lo_k)
        hi_k = jnp.where(ge_k, hi_k, mid_k)
        ge_p = cp >= kt_p
        lo_p = jnp.where(ge_p, mid_p, lo_p)
        hi_p = jnp.where(ge_p, hi_p, mid_p)
        return lo_k, hi_k, n_k, lo_p, hi_p

    ones = jnp.ones((_ROWS, 1), f32)
    st0 = (-1.001 * ones, 1.001 * ones, float(_K) * ones, -1.001 * ones, 1.001 * ones)
    lo_k, _, n_k, v_p, _ = jax.lax.fori_loop(0, _BITERS, _bis, st0)

    def _p1_body(c, s):
        v = sims_ref[:, pl.ds(c * _CHUNK, _CHUNK)]
        return s + _red(jnp.where(v > lo_k, v, 0.0), ones_c)

    p1 = jax.lax.fori_loop(0, _NFULL, _p1_body, jnp.zeros((_ROWS, 1), f32))
    v = sims_ref[:, _TAIL0:]
    p1 = p1 + _red(jnp.where(v > lo_k, v, 0.0), ones_t)

    excess = n_k - float(_NB)
    vbar = (p1 - excess * lo_k) / float(_NB)

    def _mom(v, ones):
        d = jnp.where(v > lo_k, v - vbar, 0.0)
        cur = d
        out = []
        for m in range(1, _M + 1):
            out.append(_red(cur, ones))
            if m < _M:
                cur = cur * d
        return tuple(out)

    def _mom_body(c, carry):
        part = _mom(sims_ref[:, pl.ds(c * _CHUNK, _CHUNK)], ones_c)
        return tuple(a + b for a, b in zip(carry, part))

    z14 = tuple(jnp.zeros((_ROWS, 1), f32) for _ in range(_M))
    cs = jax.lax.fori_loop(0, _NFULL, _mom_body, z14)
    cs = tuple(a + b for a, b in zip(cs, _mom(sims_ref[:, _TAIL0:], ones_t)))
    dlo = lo_k - vbar
    corr = dlo
    cs_c = [None] * (_M + 1)
    cs_c[0] = float(_NB) * jnp.ones((_ROWS, 1), f32)
    for m in range(1, _M + 1):
        cs_c[m] = cs[m - 1] - excess * corr
        corr = corr * dlo

    def _sm(u):
        t0 = jnp.zeros((_ROWS, 1), f32)
        t1 = jnp.zeros((_ROWS, 1), f32)
        cm = jnp.ones((_ROWS, 1), f32)
        for m in range(_M + 1):
            t0 = t0 + cm * cs_c[m]
            if m < _M:
                t1 = t1 + cm * cs_c[m + 1]
            cm = cm * u / float(m + 1)
        e = jnp.exp(u * vbar)
        return e * t0, e * (vbar * t0 + t1)

    def _entropy(u):
        s, mv = _sm(u)
        ep = jnp.exp(u * v_p)
        sp = s - ep
        mp = mv - v_p * ep
        return jnp.log(sp) - u * mp / sp - _EPS_H

    centers = 5.0 * jnp.ones((_ROWS, 1), f32)
    scale = 2.5
    for _ in range(13):
        h = _entropy(1.0 / centers)
        ind = 2.0 * jnp.where(h < _TARGET_ENTROPY, 1.0, 0.0) - 1.0
        centers = centers + scale * ind
        scale = scale * 0.5
    u_f = 1.0 / centers
    h_f = _entropy(u_f)

    v = sims_ref[:, 0:_NB]
    lane = jax.lax.broadcasted_iota(jnp.int32, (_ROWS, _NB), 1).astype(f32)
    pos = jnp.sum(jnp.where(lane == p, v, 0.0), axis=1, keepdims=True)

    s_f, _ = _sm(u_f)
    denom = jnp.exp(-u_f) * s_f
    cond = jnp.exp((pos - 1.0) * u_f) / denom
    ll = jnp.log(cond + 1e-7)

    vec = jnp.concatenate(
        [jnp.sum(ll, axis=0, keepdims=True),
         jnp.sum(centers, axis=0, keepdims=True),
         jnp.sum(h_f, axis=0, keepdims=True)], axis=1)
    acc_ref[...] = jnp.where(i == 0, vec, acc_ref[...] + vec)


def kernel(points, point_indices, memory_bank):
    bank_t = memory_bank.T
    pidx_f = point_indices.astype(jnp.float32).reshape(_B, 1)
    sims, acc = pl.pallas_call(
        _fused_body,
        grid=(_B // _ROWS,),
        in_specs=[
            pl.BlockSpec((_ROWS, _D), lambda i: (i, 0)),
            pl.BlockSpec((_ROWS, 1), lambda i: (i, 0)),
            pl.BlockSpec((_D, _K), lambda i: (0, 0)),
        ],
        out_specs=[
            pl.BlockSpec((_ROWS, _K), lambda i: (i, 0)),
            pl.BlockSpec((1, 3), lambda i: (0, 0)),
        ],
        out_shape=[
            jax.ShapeDtypeStruct((_B, _K), jnp.float32),
            jax.ShapeDtypeStruct((1, 3), jnp.float32),
        ],
    )(points, pidx_f, bank_t)
    inv_b = 1.0 / float(_B)
    loss = -acc[0, 0] * inv_b
    return loss, sims, acc[0, 1] * inv_b, acc[0, 2] * inv_b


# speedup vs baseline: 4.2641x; 1.2398x over previous
import jax
import jax.numpy as jnp
from jax.experimental import pallas as pl

_TARGET_ENTROPY = 8.0
_NB = 4096
_B = 1024
_D = 16
_K = 100000
_ROWS = 32
_CHUNK = 16384
_NFULL = _K // _CHUNK
_TAIL0 = _NFULL * _CHUNK
_BITERS = 13
_M = 14
_TAILW = _K - _TAIL0
_EPS_H = 4095e-7


def _fused_body(pts_ref, pidx_ref, bank_ref, sims_ref, acc_ref):
    i = pl.program_id(0)
    f32 = jnp.float32

    x = pts_ref[...]
    xn = x * jax.lax.rsqrt(jnp.sum(x * x, axis=1, keepdims=True))

    def _mm(w):
        wn = w * jax.lax.rsqrt(jnp.sum(w * w, axis=0, keepdims=True))
        return jax.lax.dot_general(
            xn, wn, (((1,), (0,)), ((), ())), preferred_element_type=f32
        )

    def _mm_chunk(c, carry):
        sims_ref[:, pl.ds(c * _CHUNK, _CHUNK)] = _mm(
            bank_ref[:, pl.ds(c * _CHUNK, _CHUNK)]
        )
        return carry

    jax.lax.fori_loop(0, _NFULL, _mm_chunk, 0)
    sims_ref[:, _TAIL0:] = _mm(bank_ref[:, _TAIL0:])

    p = pidx_ref[...]
    kt_p = p + 1.0

    ones_c = jnp.ones((_CHUNK, 8), f32)
    ones_t = jnp.ones((_TAILW, 8), f32)

    def _red(x, ones):
        return jnp.sum(x, axis=1, keepdims=True)

    def _counts(mid_k, mid_p):
        def body(c, carry):
            ck, cp = carry
            v = sims_ref[:, pl.ds(c * _CHUNK, _CHUNK)]
            ck = ck + _red(jnp.where(v > mid_k, 1.0, 0.0), ones_c)
            cp = cp + _red(jnp.where(v > mid_p, 1.0, 0.0), ones_c)
            return ck, cp
        z = jnp.zeros((_ROWS, 1), f32)
        ck, cp = jax.lax.fori_loop(0, _NFULL, body, (z, z))
        v = sims_ref[:, _TAIL0:]
        ck = ck + _red(jnp.where(v > mid_k, 1.0, 0.0), ones_t)
        cp = cp + _red(jnp.where(v > mid_p, 1.0, 0.0), ones_t)
        return ck, cp

    def _bis(_, st):
        lo_k, hi_k, n_k, lo_p, hi_p = st
        mid_k = 0.5 * (lo_k + hi_k)
        mid_p = 0.5 * (lo_p + hi_p)
        ck, cp = _counts(mid_k, mid_p)
        ge_k = ck >= float(_NB)
        n_k = jnp.where(ge_k, ck, n_k)
        lo_k = jnp.where(ge_k, mid_k, lo_k)
        hi_k = jnp.where(ge_k, hi_k, mid_k)
        ge_p = cp >= kt_p
        lo_p = jnp.where(ge_p, mid_p, lo_p)
        hi_p = jnp.where(ge_p, hi_p, mid_p)
        return lo_k, hi_k, n_k, lo_p, hi_p

    ones = jnp.ones((_ROWS, 1), f32)
    st0 = (-1.001 * ones, 1.001 * ones, float(_K) * ones, -1.001 * ones, 1.001 * ones)
    lo_k, _, n_k, v_p, _ = jax.lax.fori_loop(0, _BITERS, _bis, st0)

    def _p1_body(c, s):
        v = sims_ref[:, pl.ds(c * _CHUNK, _CHUNK)]
        return s + _red(jnp.where(v > lo_k, v, 0.0), ones_c)

    p1 = jax.lax.fori_loop(0, _NFULL, _p1_body, jnp.zeros((_ROWS, 1), f32))
    v = sims_ref[:, _TAIL0:]
    p1 = p1 + _red(jnp.where(v > lo_k, v, 0.0), ones_t)

    excess = n_k - float(_NB)
    vbar = (p1 - excess * lo_k) / float(_NB)

    def _mom(v, ones):
        d = jnp.where(v > lo_k, v - vbar, 0.0)
        cur = d
        out = []
        for m in range(1, _M + 1):
            out.append(_red(cur, ones))
            if m < _M:
                cur = cur * d
        return tuple(out)

    def _mom_body(c, carry):
        part = _mom(sims_ref[:, pl.ds(c * _CHUNK, _CHUNK)], ones_c)
        return tuple(a + b for a, b in zip(carry, part))

    z14 = tuple(jnp.zeros((_ROWS, 1), f32) for _ in range(_M))
    cs = jax.lax.fori_loop(0, _NFULL, _mom_body, z14)
    cs = tuple(a + b for a, b in zip(cs, _mom(sims_ref[:, _TAIL0:], ones_t)))
    dlo = lo_k - vbar
    corr = dlo
    cs_c = [None] * (_M + 1)
    cs_c[0] = float(_NB) * jnp.ones((_ROWS, 1), f32)
    for m in range(1, _M + 1):
        cs_c[m] = cs[m - 1] - excess * corr
        corr = corr * dlo

    def _sm(u):
        t0 = jnp.zeros((_ROWS, 1), f32)
        t1 = jnp.zeros((_ROWS, 1), f32)
        cm = jnp.ones((_ROWS, 1), f32)
        for m in range(_M + 1):
            t0 = t0 + cm * cs_c[m]
            if m < _M:
                t1 = t1 + cm * cs_c[m + 1]
            cm = cm * u / float(m + 1)
        e = jnp.exp(u * vbar)
        return e * t0, e * (vbar * t0 + t1)

    def _entropy(u):
        s, mv = _sm(u)
        ep = jnp.exp(u * v_p)
        sp = s - ep
        mp = mv - v_p * ep
        return jnp.log(sp) - u * mp / sp - _EPS_H

    centers = 5.0 * jnp.ones((_ROWS, 1), f32)
    scale = 2.5
    for _ in range(13):
        h = _entropy(1.0 / centers)
        ind = 2.0 * jnp.where(h < _TARGET_ENTROPY, 1.0, 0.0) - 1.0
        centers = centers + scale * ind
        scale = scale * 0.5
    u_f = 1.0 / centers
    h_f = _entropy(u_f)

    v = sims_ref[:, 0:_NB]
    lane = jax.lax.broadcasted_iota(jnp.int32, (_ROWS, _NB), 1).astype(f32)
    pos = jnp.sum(jnp.where(lane == p, v, 0.0), axis=1, keepdims=True)

    s_f, _ = _sm(u_f)
    denom = jnp.exp(-u_f) * s_f
    cond = jnp.exp((pos - 1.0) * u_f) / denom
    ll = jnp.log(cond + 1e-7)

    vec = jnp.concatenate(
        [jnp.sum(ll, axis=0, keepdims=True),
         jnp.sum(centers, axis=0, keepdims=True),
         jnp.sum(h_f, axis=0, keepdims=True)], axis=1)
    acc_ref[...] = jnp.where(i == 0, vec, acc_ref[...] + vec)


def kernel(points, point_indices, memory_bank):
    bank_t = memory_bank.T
    pidx_f = point_indices.astype(jnp.float32).reshape(_B, 1)
    sims, acc = pl.pallas_call(
        _fused_body,
        grid=(_B // _ROWS,),
        in_specs=[
            pl.BlockSpec((_ROWS, _D), lambda i: (i, 0)),
            pl.BlockSpec((_ROWS, 1), lambda i: (i, 0)),
            pl.BlockSpec((_D, _K), lambda i: (0, 0)),
        ],
        out_specs=[
            pl.BlockSpec((_ROWS, _K), lambda i: (i, 0)),
            pl.BlockSpec((1, 3), lambda i: (0, 0)),
        ],
        out_shape=[
            jax.ShapeDtypeStruct((_B, _K), jnp.float32),
            jax.ShapeDtypeStruct((1, 3), jnp.float32),
        ],
    )(points, pidx_f, bank_t)
    inv_b = 1.0 / float(_B)
    loss = -acc[0, 0] * inv_b
    return loss, sims, acc[0, 1] * inv_b, acc[0, 2] * inv_b


# speedup vs baseline: 4.4876x; 1.0524x over previous
import jax
import jax.numpy as jnp
from jax.experimental import pallas as pl

_TARGET_ENTROPY = 8.0
_NB = 4096
_B = 1024
_D = 16
_K = 100000
_ROWS = 32
_CHUNK = 32768
_NFULL = _K // _CHUNK
_TAIL0 = _NFULL * _CHUNK
_BITERS = 13
_M = 14
_TAILW = _K - _TAIL0
_EPS_H = 4095e-7


def _fused_body(pts_ref, pidx_ref, bank_ref, sims_ref, acc_ref):
    i = pl.program_id(0)
    f32 = jnp.float32

    x = pts_ref[...]
    xn = x * jax.lax.rsqrt(jnp.sum(x * x, axis=1, keepdims=True))

    def _mm(w):
        wn = w * jax.lax.rsqrt(jnp.sum(w * w, axis=0, keepdims=True))
        return jax.lax.dot_general(
            xn, wn, (((1,), (0,)), ((), ())), preferred_element_type=f32
        )

    def _mm_chunk(c, carry):
        sims_ref[:, pl.ds(c * _CHUNK, _CHUNK)] = _mm(
            bank_ref[:, pl.ds(c * _CHUNK, _CHUNK)]
        )
        return carry

    jax.lax.fori_loop(0, _NFULL, _mm_chunk, 0)
    sims_ref[:, _TAIL0:] = _mm(bank_ref[:, _TAIL0:])

    p = pidx_ref[...]
    kt_p = p + 1.0

    ones_c = jnp.ones((_CHUNK, 8), f32)
    ones_t = jnp.ones((_TAILW, 8), f32)

    def _red(x, ones):
        return jnp.sum(x, axis=1, keepdims=True)

    def _counts(mid_k, mid_p):
        def body(c, carry):
            ck, cp = carry
            v = sims_ref[:, pl.ds(c * _CHUNK, _CHUNK)]
            ck = ck + _red((v > mid_k).astype(f32), ones_c)
            cp = cp + _red((v > mid_p).astype(f32), ones_c)
            return ck, cp
        z = jnp.zeros((_ROWS, 1), f32)
        ck, cp = jax.lax.fori_loop(0, _NFULL, body, (z, z))
        v = sims_ref[:, _TAIL0:]
        ck = ck + _red((v > mid_k).astype(f32), ones_t)
        cp = cp + _red((v > mid_p).astype(f32), ones_t)
        return ck, cp

    def _bis(_, st):
        lo_k, hi_k, n_k, lo_p, hi_p = st
        mid_k = 0.5 * (lo_k + hi_k)
        mid_p = 0.5 * (lo_p + hi_p)
        ck, cp = _counts(mid_k, mid_p)
        ge_k = ck >= float(_NB)
        n_k = jnp.where(ge_k, ck, n_k)
        lo_k = jnp.where(ge_k, mid_k, lo_k)
        hi_k = jnp.where(ge_k, hi_k, mid_k)
        ge_p = cp >= kt_p
        lo_p = jnp.where(ge_p, mid_p, lo_p)
        hi_p = jnp.where(ge_p, hi_p, mid_p)
        return lo_k, hi_k, n_k, lo_p, hi_p

    ones = jnp.ones((_ROWS, 1), f32)
    st0 = (-1.001 * ones, 1.001 * ones, float(_K) * ones, -1.001 * ones, 1.001 * ones)
    lo_k, hi_k, n_k, lo_p, hi_p = jax.lax.fori_loop(0, _BITERS - 1, _bis, st0)

    mid_k = 0.5 * (lo_k + hi_k)
    mid_p = 0.5 * (lo_p + hi_p)

    def _fin_body(c, carry):
        ck, cp, ms = carry
        v = sims_ref[:, pl.ds(c * _CHUNK, _CHUNK)]
        mk = v > mid_k
        ck = ck + jnp.sum(mk.astype(f32), axis=1, keepdims=True)
        cp = cp + jnp.sum((v > mid_p).astype(f32), axis=1, keepdims=True)
        ms = ms + jnp.sum(jnp.where(mk, v, 0.0), axis=1, keepdims=True)
        return ck, cp, ms

    zf = jnp.zeros((_ROWS, 1), f32)
    ck, cp, ms = jax.lax.fori_loop(0, _NFULL, _fin_body, (zf, zf, zf))
    v = sims_ref[:, _TAIL0:]
    mk = v > mid_k
    ck = ck + jnp.sum(mk.astype(f32), axis=1, keepdims=True)
    cp = cp + jnp.sum((v > mid_p).astype(f32), axis=1, keepdims=True)
    ms = ms + jnp.sum(jnp.where(mk, v, 0.0), axis=1, keepdims=True)
    ge_k = ck >= float(_NB)
    n_k = jnp.where(ge_k, ck, n_k)
    lo_k = jnp.where(ge_k, mid_k, lo_k)
    v_p = jnp.where(cp >= kt_p, mid_p, lo_p)

    excess = n_k - float(_NB)
    vbar = ms / jnp.maximum(ck, 1.0)

    def _mom(v, ones):
        d = jnp.where(v > lo_k, v - vbar, 0.0)
        cur = d
        out = []
        for m in range(1, _M + 1):
            out.append(_red(cur, ones))
            if m < _M:
                cur = cur * d
        return tuple(out)

    def _mom_body(c, carry):
        part = _mom(sims_ref[:, pl.ds(c * _CHUNK, _CHUNK)], ones_c)
        return tuple(a + b for a, b in zip(carry, part))

    z14 = tuple(jnp.zeros((_ROWS, 1), f32) for _ in range(_M))
    cs = jax.lax.fori_loop(0, _NFULL, _mom_body, z14)
    cs = tuple(a + b for a, b in zip(cs, _mom(sims_ref[:, _TAIL0:], ones_t)))
    dlo = lo_k - vbar
    corr = dlo
    cs_c = [None] * (_M + 1)
    cs_c[0] = float(_NB) * jnp.ones((_ROWS, 1), f32)
    for m in range(1, _M + 1):
        cs_c[m] = cs[m - 1] - excess * corr
        corr = corr * dlo

    def _sm(u):
        t0 = jnp.zeros((_ROWS, 1), f32)
        t1 = jnp.zeros((_ROWS, 1), f32)
        cm = jnp.ones((_ROWS, 1), f32)
        for m in range(_M + 1):
            t0 = t0 + cm * cs_c[m]
            if m < _M:
                t1 = t1 + cm * cs_c[m + 1]
            cm = cm * u / float(m + 1)
        e = jnp.exp(u * vbar)
        return e * t0, e * (vbar * t0 + t1)

    def _entropy(u):
        s, mv = _sm(u)
        ep = jnp.exp(u * v_p)
        sp = s - ep
        mp = mv - v_p * ep
        return jnp.log(sp) - u * mp / sp - _EPS_H

    centers = 5.0 * jnp.ones((_ROWS, 1), f32)
    scale = 2.5
    for _ in range(13):
        h = _entropy(1.0 / centers)
        ind = 2.0 * jnp.where(h < _TARGET_ENTROPY, 1.0, 0.0) - 1.0
        centers = centers + scale * ind
        scale = scale * 0.5
    u_f = 1.0 / centers
    h_f = _entropy(u_f)

    v = sims_ref[:, 0:_NB]
    lane = jax.lax.broadcasted_iota(jnp.int32, (_ROWS, _NB), 1).astype(f32)
    pos = jnp.sum(jnp.where(lane == p, v, 0.0), axis=1, keepdims=True)

    s_f, _ = _sm(u_f)
    denom = jnp.exp(-u_f) * s_f
    cond = jnp.exp((pos - 1.0) * u_f) / denom
    ll = jnp.log(cond + 1e-7)

    vec = jnp.concatenate(
        [jnp.sum(ll, axis=0, keepdims=True),
         jnp.sum(centers, axis=0, keepdims=True),
         jnp.sum(h_f, axis=0, keepdims=True)], axis=1)
    acc_ref[...] = jnp.where(i == 0, vec, acc_ref[...] + vec)


def kernel(points, point_indices, memory_bank):
    bank_t = memory_bank.T
    pidx_f = point_indices.astype(jnp.float32).reshape(_B, 1)
    sims, acc = pl.pallas_call(
        _fused_body,
        grid=(_B // _ROWS,),
        in_specs=[
            pl.BlockSpec((_ROWS, _D), lambda i: (i, 0)),
            pl.BlockSpec((_ROWS, 1), lambda i: (i, 0)),
            pl.BlockSpec((_D, _K), lambda i: (0, 0)),
        ],
        out_specs=[
            pl.BlockSpec((_ROWS, _K), lambda i: (i, 0)),
            pl.BlockSpec((1, 3), lambda i: (0, 0)),
        ],
        out_shape=[
            jax.ShapeDtypeStruct((_B, _K), jnp.float32),
            jax.ShapeDtypeStruct((1, 3), jnp.float32),
        ],
    )(points, pidx_f, bank_t)
    inv_b = 1.0 / float(_B)
    loss = -acc[0, 0] * inv_b
    return loss, sims, acc[0, 1] * inv_b, acc[0, 2] * inv_b


# speedup vs baseline: 4.6923x; 1.0456x over previous
import jax
import jax.numpy as jnp
from jax.experimental import pallas as pl

_TARGET_ENTROPY = 8.0
_NB = 4096
_B = 1024
_D = 16
_K = 100000
_ROWS = 32
_CHUNK = 32768
_NFULL = _K // _CHUNK
_TAIL0 = _NFULL * _CHUNK
_BITERS = 12
_M = 14
_TAILW = _K - _TAIL0
_EPS_H = 4095e-7


def _fused_body(pts_ref, pidx_ref, bank_ref, sims_ref, acc_ref):
    i = pl.program_id(0)
    f32 = jnp.float32

    x = pts_ref[...]
    xn = x * jax.lax.rsqrt(jnp.sum(x * x, axis=1, keepdims=True))

    def _mm(w):
        wn = w * jax.lax.rsqrt(jnp.sum(w * w, axis=0, keepdims=True))
        return jax.lax.dot_general(
            xn, wn, (((1,), (0,)), ((), ())), preferred_element_type=f32
        )

    def _mm_chunk(c, carry):
        sims_ref[:, pl.ds(c * _CHUNK, _CHUNK)] = _mm(
            bank_ref[:, pl.ds(c * _CHUNK, _CHUNK)]
        )
        return carry

    jax.lax.fori_loop(0, _NFULL, _mm_chunk, 0)
    sims_ref[:, _TAIL0:] = _mm(bank_ref[:, _TAIL0:])

    p = pidx_ref[...]
    kt_p = p + 1.0

    ones_c = jnp.ones((_CHUNK, 8), f32)
    ones_t = jnp.ones((_TAILW, 8), f32)

    def _red(x, ones):
        return jnp.sum(x, axis=1, keepdims=True)

    def _counts(mid_k, mid_p):
        def body(c, carry):
            ck, cp = carry
            v = sims_ref[:, pl.ds(c * _CHUNK, _CHUNK)]
            ck = ck + _red((v > mid_k).astype(f32), ones_c)
            cp = cp + _red((v > mid_p).astype(f32), ones_c)
            return ck, cp
        z = jnp.zeros((_ROWS, 1), f32)
        ck, cp = jax.lax.fori_loop(0, _NFULL, body, (z, z))
        v = sims_ref[:, _TAIL0:]
        ck = ck + _red((v > mid_k).astype(f32), ones_t)
        cp = cp + _red((v > mid_p).astype(f32), ones_t)
        return ck, cp

    def _bis(_, st):
        lo_k, hi_k, n_k, lo_p, hi_p = st
        mid_k = 0.5 * (lo_k + hi_k)
        mid_p = 0.5 * (lo_p + hi_p)
        ck, cp = _counts(mid_k, mid_p)
        ge_k = ck >= float(_NB)
        n_k = jnp.where(ge_k, ck, n_k)
        lo_k = jnp.where(ge_k, mid_k, lo_k)
        hi_k = jnp.where(ge_k, hi_k, mid_k)
        ge_p = cp >= kt_p
        lo_p = jnp.where(ge_p, mid_p, lo_p)
        hi_p = jnp.where(ge_p, hi_p, mid_p)
        return lo_k, hi_k, n_k, lo_p, hi_p

    ones = jnp.ones((_ROWS, 1), f32)
    st0 = (-1.001 * ones, 1.001 * ones, float(_K) * ones, -1.001 * ones, 1.001 * ones)
    lo_k, hi_k, n_k, lo_p, hi_p = jax.lax.fori_loop(0, _BITERS - 1, _bis, st0)

    mid_k = 0.5 * (lo_k + hi_k)
    mid_p = 0.5 * (lo_p + hi_p)

    def _fin_body(c, carry):
        ck, cp, ms = carry
        v = sims_ref[:, pl.ds(c * _CHUNK, _CHUNK)]
        mk = v > mid_k
        ck = ck + jnp.sum(mk.astype(f32), axis=1, keepdims=True)
        cp = cp + jnp.sum((v > mid_p).astype(f32), axis=1, keepdims=True)
        ms = ms + jnp.sum(jnp.where(mk, v, 0.0), axis=1, keepdims=True)
        return ck, cp, ms

    zf = jnp.zeros((_ROWS, 1), f32)
    ck, cp, ms = jax.lax.fori_loop(0, _NFULL, _fin_body, (zf, zf, zf))
    v = sims_ref[:, _TAIL0:]
    mk = v > mid_k
    ck = ck + jnp.sum(mk.astype(f32), axis=1, keepdims=True)
    cp = cp + jnp.sum((v > mid_p).astype(f32), axis=1, keepdims=True)
    ms = ms + jnp.sum(jnp.where(mk, v, 0.0), axis=1, keepdims=True)
    ge_k = ck >= float(_NB)
    n_k = jnp.where(ge_k, ck, n_k)
    lo_k = jnp.where(ge_k, mid_k, lo_k)
    v_p = jnp.where(cp >= kt_p, mid_p, lo_p)

    excess = n_k - float(_NB)
    vbar = ms / jnp.maximum(ck, 1.0)

    def _mom(v, ones):
        d = jnp.where(v > lo_k, v - vbar, 0.0)
        cur = d
        out = []
        for m in range(1, _M + 1):
            out.append(_red(cur, ones))
            if m < _M:
                cur = cur * d
        return tuple(out)

    def _mom_body(c, carry):
        part = _mom(sims_ref[:, pl.ds(c * _CHUNK, _CHUNK)], ones_c)
        return tuple(a + b for a, b in zip(carry, part))

    z14 = tuple(jnp.zeros((_ROWS, 1), f32) for _ in range(_M))
    cs = jax.lax.fori_loop(0, _NFULL, _mom_body, z14)
    cs = tuple(a + b for a, b in zip(cs, _mom(sims_ref[:, _TAIL0:], ones_t)))
    dlo = lo_k - vbar
    corr = dlo
    cs_c = [None] * (_M + 1)
    cs_c[0] = float(_NB) * jnp.ones((_ROWS, 1), f32)
    for m in range(1, _M + 1):
        cs_c[m] = cs[m - 1] - excess * corr
        corr = corr * dlo

    def _sm(u):
        t0 = jnp.zeros((_ROWS, 1), f32)
        t1 = jnp.zeros((_ROWS, 1), f32)
        cm = jnp.ones((_ROWS, 1), f32)
        for m in range(_M + 1):
            t0 = t0 + cm * cs_c[m]
            if m < _M:
                t1 = t1 + cm * cs_c[m + 1]
            cm = cm * u / float(m + 1)
        e = jnp.exp(u * vbar)
        return e * t0, e * (vbar * t0 + t1)

    def _entropy(u):
        s, mv = _sm(u)
        ep = jnp.exp(u * v_p)
        sp = s - ep
        mp = mv - v_p * ep
        return jnp.log(sp) - u * mp / sp - _EPS_H

    centers = 5.0 * jnp.ones((_ROWS, 1), f32)
    scale = 2.5
    for _ in range(13):
        h = _entropy(1.0 / centers)
        ind = 2.0 * jnp.where(h < _TARGET_ENTROPY, 1.0, 0.0) - 1.0
        centers = centers + scale * ind
        scale = scale * 0.5
    u_f = 1.0 / centers
    h_f = _entropy(u_f)

    v = sims_ref[:, 0:_NB]
    lane = jax.lax.broadcasted_iota(jnp.int32, (_ROWS, _NB), 1).astype(f32)
    pos = jnp.sum(jnp.where(lane == p, v, 0.0), axis=1, keepdims=True)

    s_f, _ = _sm(u_f)
    denom = jnp.exp(-u_f) * s_f
    cond = jnp.exp((pos - 1.0) * u_f) / denom
    ll = jnp.log(cond + 1e-7)

    vec = jnp.concatenate(
        [jnp.sum(ll, axis=0, keepdims=True),
         jnp.sum(centers, axis=0, keepdims=True),
         jnp.sum(h_f, axis=0, keepdims=True)], axis=1)
    acc_ref[...] = jnp.where(i == 0, vec, acc_ref[...] + vec)


def kernel(points, point_indices, memory_bank):
    bank_t = memory_bank.T
    pidx_f = point_indices.astype(jnp.float32).reshape(_B, 1)
    sims, acc = pl.pallas_call(
        _fused_body,
        grid=(_B // _ROWS,),
        in_specs=[
            pl.BlockSpec((_ROWS, _D), lambda i: (i, 0)),
            pl.BlockSpec((_ROWS, 1), lambda i: (i, 0)),
            pl.BlockSpec((_D, _K), lambda i: (0, 0)),
        ],
        out_specs=[
            pl.BlockSpec((_ROWS, _K), lambda i: (i, 0)),
            pl.BlockSpec((1, 3), lambda i: (0, 0)),
        ],
        out_shape=[
            jax.ShapeDtypeStruct((_B, _K), jnp.float32),
            jax.ShapeDtypeStruct((1, 3), jnp.float32),
        ],
    )(points, pidx_f, bank_t)
    inv_b = 1.0 / float(_B)
    loss = -acc[0, 0] * inv_b
    return loss, sims, acc[0, 1] * inv_b, acc[0, 2] * inv_b
